# baseline (speedup 1.0000x reference)
import functools

import jax
import jax.numpy as jnp
from jax import lax
from jax.experimental import pallas as pl
from jax.experimental.pallas import tpu as pltpu
from jax.experimental.pallas import tpu_sc as plsc

NC = 2
NS = 16
L = 16
NW = NC * NS


def _make_emb_mean(V, D, S, B):
  assert B % NW == 0
  b_per_w = B // NW
  assert b_per_w % 8 == 0
  chunks = b_per_w * D // L

  mesh = plsc.VectorSubcoreMesh(core_axis_name="c", subcore_axis_name="s")

  @functools.partial(
      pl.kernel,
      mesh=mesh,
      out_type=jax.ShapeDtypeStruct((B, D), jnp.float32),
      compiler_params=pltpu.CompilerParams(use_tc_tiling_on_sc=False),
      scratch_types=[
          pltpu.VMEM((S, b_per_w), jnp.int32),
          pltpu.VMEM((b_per_w, D), jnp.float32),
          pltpu.VMEM((b_per_w, D), jnp.float32),
          pltpu.VMEM((b_per_w,), jnp.int32),
          pltpu.VMEM_SHARED((NS * b_per_w, D), jnp.float32),
          pltpu.SemaphoreType.DMA,
          pltpu.SemaphoreType.DMA,
      ],
  )
  def emb_mean(table_hbm, idx_hbm, out_hbm, idx_v, rows0, rows1, ramp_v,
               acc_sh, sem0, sem1):
    cid = lax.axis_index("c")
    sid = lax.axis_index("s")
    wid = cid * NS + sid
    base_local = sid * b_per_w
    base_glob = wid * b_per_w

    pltpu.sync_copy(idx_hbm.at[wid], idx_v)

    for i in range(b_per_w // L):
      ramp_v[pl.ds(i * L, L)] = (
          lax.iota(jnp.int32, L) + (base_local + i * L))

    bufs = (rows0, rows1)
    sems = (sem0, sem1)

    def start(s, p):
      pltpu.async_copy(table_hbm.at[idx_v.at[s]], bufs[p], sems[p])

    def wait(p):
      pltpu.make_async_copy(table_hbm.at[idx_v.at[0]], bufs[p], sems[p]).wait()

    start(0, 0)
    start(1, 1)
    wait(0)
    pltpu.sync_copy(rows0, acc_sh.at[ramp_v])
    wait(1)
    pltpu.sync_copy(rows1, acc_sh.at[ramp_v], add=True)

    def body(k, carry):
      s = 2 * k
      for p in range(2):
        start(s + p, p)
      for p in range(2):
        wait(p)
        pltpu.sync_copy(bufs[p], acc_sh.at[ramp_v], add=True)
      return carry

    lax.fori_loop(1, S // 2, body, 0)

    pltpu.sync_copy(acc_sh.at[pl.ds(base_local, b_per_w)], rows0)
    inv = jnp.float32(1.0 / S)

    def scale_body(b, carry):
      for c in range(D // L):
        rows0[b, pl.ds(c * L, L)] = rows0[b, pl.ds(c * L, L)] * inv
      return carry

    lax.fori_loop(0, b_per_w, scale_body, 0)
    pltpu.sync_copy(rows0, out_hbm.at[pl.ds(base_glob, b_per_w)])

  return emb_mean


def kernel(embedding_weight, input_seqs):
  V, D = embedding_weight.shape
  S, B = input_seqs.shape
  idx = input_seqs.astype(jnp.int32)
  idx = idx.reshape(S, NW, B // NW).transpose(1, 0, 2)
  return _make_emb_mean(V, D, S, B)(embedding_weight, idx)
